# baseline (speedup 1.0000x reference)
import functools

import jax
import jax.numpy as jnp
from jax import lax
from jax.experimental import pallas as pl
from jax.experimental.pallas import tpu as pltpu
from jax.experimental.pallas import tpu_sc as plsc

_L = 16
_NC = 2
_NS = 16
_NW = _NC * _NS
_ICHUNK = 128


def _tc_body(pt_ref, nt_ref, w_ref, p_ref, n_ref):
    h = pt_ref.shape[1]
    w = w_ref[...]
    p_ref[...] = jnp.sum(pt_ref[...] * w[:, :h], axis=1)
    n_ref[...] = jnp.sum(nt_ref[...] * w[:, h:], axis=1)


@functools.lru_cache(maxsize=None)
def _build_tc(R, H, BR):
    grid = ((R + BR - 1) // BR,)
    return pl.pallas_call(
        _tc_body,
        grid=grid,
        in_specs=[
            pl.BlockSpec((BR, H), lambda i: (i, 0)),
            pl.BlockSpec((BR, H), lambda i: (i, 0)),
            pl.BlockSpec((1, 2 * H), lambda i: (0, 0)),
        ],
        out_specs=[
            pl.BlockSpec((BR,), lambda i: (i,)),
            pl.BlockSpec((BR,), lambda i: (i,)),
        ],
        out_shape=[
            jax.ShapeDtypeStruct((R,), jnp.float32),
            jax.ShapeDtypeStruct((R,), jnp.float32),
        ],
    )


@functools.lru_cache(maxsize=None)
def _build_sc(B):
    bpw = B // _NW
    n_ichunks = bpw // _ICHUNK
    mesh = plsc.VectorSubcoreMesh(core_axis_name="c", subcore_axis_name="s")

    @functools.partial(
        pl.kernel,
        mesh=mesh,
        out_type=jax.ShapeDtypeStruct((B,), jnp.float32),
        compiler_params=pltpu.CompilerParams(
            needs_layout_passes=False, use_tc_tiling_on_sc=False
        ),
        scratch_types=[
            pltpu.VMEM((2 * bpw,), jnp.int32),
            pltpu.VMEM((bpw,), jnp.int32),
            pltpu.VMEM((bpw,), jnp.int32),
            pltpu.VMEM((bpw,), jnp.float32),
            pltpu.VMEM((bpw,), jnp.float32),
            pltpu.VMEM((_L,), jnp.float32),
            pltpu.VMEM((bpw,), jnp.float32),
            pltpu.SemaphoreType.DMA,
        ],
    )
    def sc_kernel(x_hbm, p_hbm, n_hbm, b_hbm, out_hbm,
                  xv, piv, niv, pv, nv, bv, ov, sem):
        wid = lax.axis_index("s") * _NC + lax.axis_index("c")
        base = wid * bpw

        pltpu.sync_copy(x_hbm.at[pl.ds(2 * base, 2 * bpw)], xv)
        pltpu.sync_copy(b_hbm, bv)

        iota = lax.iota(jnp.int32, _L)

        def deint(g, carry):
            strided = 2 * (g * _L + iota)
            piv[pl.ds(g * _L, _L)] = plsc.load_gather(xv, [strided])
            niv[pl.ds(g * _L, _L)] = plsc.load_gather(xv, [strided + 1])
            return carry
        lax.fori_loop(0, bpw // _L, deint, 0)

        copies = []
        for c in range(n_ichunks):
            sl = pl.ds(c * _ICHUNK, _ICHUNK)
            copies.append(pltpu.async_copy(p_hbm.at[piv.at[sl]], pv.at[sl], sem))
            copies.append(pltpu.async_copy(n_hbm.at[niv.at[sl]], nv.at[sl], sem))
        for cp in copies:
            cp.wait()

        b_vec = bv[...]

        def add_body(g, carry):
            sl = pl.ds(g * _L, _L)
            ov[sl] = pv[sl] + nv[sl] + b_vec
            return carry
        lax.fori_loop(0, bpw // _L, add_body, 0)

        pltpu.sync_copy(ov, out_hbm.at[pl.ds(base, bpw)])

    return sc_kernel


def kernel(x, emb_proton, emb_neutron, W, b):
    B = x.shape[0]
    R, H = emb_proton.shape
    p, n = _build_tc(R, H, 4096)(emb_proton, emb_neutron, W.reshape(1, 2 * H))
    b_vec = jnp.broadcast_to(b.reshape(()), (_L,)).astype(jnp.float32)
    out = _build_sc(B)(x.reshape(-1), p, n, b_vec)
    return out.reshape(B, 1)


# speedup vs baseline: 1.3794x; 1.3794x over previous
import functools

import jax
import jax.numpy as jnp
from jax import lax
from jax.experimental import pallas as pl
from jax.experimental.pallas import tpu as pltpu
from jax.experimental.pallas import tpu_sc as plsc

_L = 16
_NC = 2
_NS = 16
_NW = _NC * _NS
_ICHUNK = 128


def _tc_body(pt_ref, nt_ref, w_ref, p_ref, n_ref):
    h = pt_ref.shape[1]
    br = pt_ref.shape[0]
    w = w_ref[...]
    dn = (((1,), (1,)), ((), ()))
    p = lax.dot_general(w[:, :h], pt_ref[...], dn,
                        preferred_element_type=jnp.float32)
    n = lax.dot_general(w[:, h:], nt_ref[...], dn,
                        preferred_element_type=jnp.float32)
    p_ref[...] = p.reshape(br)
    n_ref[...] = n.reshape(br)


@functools.lru_cache(maxsize=None)
def _build_tc(R, H, BR):
    nblk = (R + BR - 1) // BR
    rpad = nblk * BR
    return pl.pallas_call(
        _tc_body,
        grid=(nblk,),
        in_specs=[
            pl.BlockSpec((BR, H), lambda i: (i, 0)),
            pl.BlockSpec((BR, H), lambda i: (i, 0)),
            pl.BlockSpec((1, 2 * H), lambda i: (0, 0)),
        ],
        out_specs=[
            pl.BlockSpec((BR,), lambda i: (i,)),
            pl.BlockSpec((BR,), lambda i: (i,)),
        ],
        out_shape=[
            jax.ShapeDtypeStruct((rpad,), jnp.float32),
            jax.ShapeDtypeStruct((rpad,), jnp.float32),
        ],
    )


@functools.lru_cache(maxsize=None)
def _build_sc(B):
    bpw = B // _NW
    n_ichunks = bpw // _ICHUNK
    mesh = plsc.VectorSubcoreMesh(core_axis_name="c", subcore_axis_name="s")

    @functools.partial(
        pl.kernel,
        mesh=mesh,
        out_type=jax.ShapeDtypeStruct((B,), jnp.float32),
        compiler_params=pltpu.CompilerParams(
            needs_layout_passes=False, use_tc_tiling_on_sc=False
        ),
        scratch_types=[
            pltpu.VMEM((2 * bpw,), jnp.int32),
            pltpu.VMEM((bpw,), jnp.int32),
            pltpu.VMEM((bpw,), jnp.int32),
            pltpu.VMEM((bpw,), jnp.float32),
            pltpu.VMEM((bpw,), jnp.float32),
            pltpu.VMEM((_L,), jnp.float32),
            pltpu.VMEM((bpw,), jnp.float32),
            pltpu.SemaphoreType.DMA,
        ],
    )
    def sc_kernel(x_hbm, p_hbm, n_hbm, b_hbm, out_hbm,
                  xv, piv, niv, pv, nv, bv, ov, sem):
        wid = lax.axis_index("s") * _NC + lax.axis_index("c")
        base = wid * bpw

        pltpu.sync_copy(x_hbm.at[pl.ds(2 * base, 2 * bpw)], xv)
        pltpu.sync_copy(b_hbm, bv)

        iota = lax.iota(jnp.int32, _L)

        def deint(g, carry):
            strided = 2 * (g * _L + iota)
            piv[pl.ds(g * _L, _L)] = plsc.load_gather(xv, [strided])
            niv[pl.ds(g * _L, _L)] = plsc.load_gather(xv, [strided + 1])
            return carry
        lax.fori_loop(0, bpw // _L, deint, 0)

        copies = []
        for c in range(n_ichunks):
            sl = pl.ds(c * _ICHUNK, _ICHUNK)
            copies.append(pltpu.async_copy(p_hbm.at[piv.at[sl]], pv.at[sl], sem))
            copies.append(pltpu.async_copy(n_hbm.at[niv.at[sl]], nv.at[sl], sem))
        for cp in copies:
            cp.wait()

        b_vec = bv[...]

        def add_body(g, carry):
            sl = pl.ds(g * _L, _L)
            ov[sl] = pv[sl] + nv[sl] + b_vec
            return carry
        lax.fori_loop(0, bpw // _L, add_body, 0)

        pltpu.sync_copy(ov, out_hbm.at[pl.ds(base, bpw)])

    return sc_kernel


def kernel(x, emb_proton, emb_neutron, W, b):
    B = x.shape[0]
    R, H = emb_proton.shape
    p, n = _build_tc(R, H, 4096)(emb_proton, emb_neutron, W.reshape(1, 2 * H))
    b_vec = jnp.broadcast_to(b.reshape(()), (_L,)).astype(jnp.float32)
    out = _build_sc(B)(x.reshape(-1), p, n, b_vec)
    return out.reshape(B, 1)


# speedup vs baseline: 4.0627x; 2.9452x over previous
import functools

import jax
import jax.numpy as jnp
from jax import lax
from jax.experimental import pallas as pl
from jax.experimental.pallas import tpu as pltpu
from jax.experimental.pallas import tpu_sc as plsc

_L = 16
_NC = 2
_NS = 16
_NW = _NC * _NS
_ICHUNK = 128


def _tc_body(ptt_ref, ntt_ref, w_ref, p_ref, n_ref):
    h = ptt_ref.shape[0]
    br = ptt_ref.shape[1]
    w = w_ref[...]
    dn = (((1,), (0,)), ((), ()))
    p = lax.dot_general(w[:, :h], ptt_ref[...], dn,
                        preferred_element_type=jnp.float32)
    n = lax.dot_general(w[:, h:], ntt_ref[...], dn,
                        preferred_element_type=jnp.float32)
    p_ref[...] = p.reshape(br)
    n_ref[...] = n.reshape(br)


@functools.lru_cache(maxsize=None)
def _build_tc(R, H, BR):
    nblk = (R + BR - 1) // BR
    rpad = nblk * BR
    return pl.pallas_call(
        _tc_body,
        grid=(nblk,),
        in_specs=[
            pl.BlockSpec((H, BR), lambda i: (0, i)),
            pl.BlockSpec((H, BR), lambda i: (0, i)),
            pl.BlockSpec((1, 2 * H), lambda i: (0, 0)),
        ],
        out_specs=[
            pl.BlockSpec((BR,), lambda i: (i,)),
            pl.BlockSpec((BR,), lambda i: (i,)),
        ],
        out_shape=[
            jax.ShapeDtypeStruct((rpad,), jnp.float32),
            jax.ShapeDtypeStruct((rpad,), jnp.float32),
        ],
    )


@functools.lru_cache(maxsize=None)
def _build_sc(B):
    bpw = B // _NW
    n_ichunks = bpw // _ICHUNK
    mesh = plsc.VectorSubcoreMesh(core_axis_name="c", subcore_axis_name="s")

    @functools.partial(
        pl.kernel,
        mesh=mesh,
        out_type=jax.ShapeDtypeStruct((B,), jnp.float32),
        compiler_params=pltpu.CompilerParams(
            needs_layout_passes=False, use_tc_tiling_on_sc=False
        ),
        scratch_types=[
            pltpu.VMEM((bpw,), jnp.int32),
            pltpu.VMEM((bpw,), jnp.int32),
            pltpu.VMEM((bpw,), jnp.float32),
            pltpu.VMEM((bpw,), jnp.float32),
            pltpu.VMEM((_L,), jnp.float32),
            pltpu.VMEM((bpw,), jnp.float32),
            pltpu.SemaphoreType.DMA,
        ],
    )
    def sc_kernel(xt_hbm, p_hbm, n_hbm, b_hbm, out_hbm,
                  piv, niv, pv, nv, bv, ov, sem):
        wid = lax.axis_index("s") * _NC + lax.axis_index("c")
        base = wid * bpw

        pltpu.sync_copy(xt_hbm.at[0, pl.ds(base, bpw)], piv)
        pltpu.sync_copy(xt_hbm.at[1, pl.ds(base, bpw)], niv)
        pltpu.sync_copy(b_hbm, bv)

        copies = []
        for c in range(n_ichunks):
            sl = pl.ds(c * _ICHUNK, _ICHUNK)
            copies.append(pltpu.async_copy(p_hbm.at[piv.at[sl]], pv.at[sl], sem))
            copies.append(pltpu.async_copy(n_hbm.at[niv.at[sl]], nv.at[sl], sem))
        for cp in copies:
            cp.wait()

        b_vec = bv[...]

        def add_body(g, carry):
            sl = pl.ds(g * _L, _L)
            ov[sl] = pv[sl] + nv[sl] + b_vec
            return carry
        lax.fori_loop(0, bpw // _L, add_body, 0)

        pltpu.sync_copy(ov, out_hbm.at[pl.ds(base, bpw)])

    return sc_kernel


def kernel(x, emb_proton, emb_neutron, W, b):
    B = x.shape[0]
    R, H = emb_proton.shape
    p, n = _build_tc(R, H, 4096)(emb_proton.T, emb_neutron.T,
                                 W.reshape(1, 2 * H))
    b_vec = jnp.broadcast_to(b.reshape(()), (_L,)).astype(jnp.float32)
    out = _build_sc(B)(x.T, p, n, b_vec)
    return out.reshape(B, 1)


# speedup vs baseline: 4.6659x; 1.1485x over previous
import functools

import jax
import jax.numpy as jnp
from jax import lax
from jax.experimental import pallas as pl
from jax.experimental.pallas import tpu as pltpu
from jax.experimental.pallas import tpu_sc as plsc

_L = 16
_NC = 2
_NS = 16
_NW = _NC * _NS
_ICHUNK = 128


def _tc_body(ptt_ref, ntt_ref, w_ref, p_ref, n_ref):
    h = ptt_ref.shape[0]
    br = ptt_ref.shape[1]
    w = w_ref[...]
    dn = (((1,), (0,)), ((), ()))
    p = lax.dot_general(w[:, :h], ptt_ref[...], dn,
                        preferred_element_type=jnp.float32)
    n = lax.dot_general(w[:, h:], ntt_ref[...], dn,
                        preferred_element_type=jnp.float32)
    p_ref[...] = p.reshape(br)
    n_ref[...] = n.reshape(br)


@functools.lru_cache(maxsize=None)
def _build_tc(R, H, BR):
    nblk = (R + BR - 1) // BR
    rpad = nblk * BR
    return pl.pallas_call(
        _tc_body,
        grid=(nblk,),
        in_specs=[
            pl.BlockSpec((H, BR), lambda i: (0, i)),
            pl.BlockSpec((H, BR), lambda i: (0, i)),
            pl.BlockSpec((1, 2 * H), lambda i: (0, 0)),
        ],
        out_specs=[
            pl.BlockSpec((BR,), lambda i: (i,)),
            pl.BlockSpec((BR,), lambda i: (i,)),
        ],
        out_shape=[
            jax.ShapeDtypeStruct((rpad,), jnp.float32),
            jax.ShapeDtypeStruct((rpad,), jnp.float32),
        ],
    )


@functools.lru_cache(maxsize=None)
def _build_sc(B):
    bpw = B // _NW
    n_ichunks = bpw // _ICHUNK
    mesh = plsc.VectorSubcoreMesh(core_axis_name="c", subcore_axis_name="s")

    @functools.partial(
        pl.kernel,
        mesh=mesh,
        out_type=jax.ShapeDtypeStruct((B,), jnp.float32),
        compiler_params=pltpu.CompilerParams(
            needs_layout_passes=False, use_tc_tiling_on_sc=False
        ),
        scratch_types=[
            pltpu.VMEM((bpw,), jnp.int32),
            pltpu.VMEM((bpw,), jnp.int32),
            pltpu.VMEM((bpw,), jnp.float32),
            pltpu.VMEM((bpw,), jnp.float32),
            pltpu.VMEM((_L,), jnp.float32),
            pltpu.VMEM((bpw,), jnp.float32),
            pltpu.SemaphoreType.DMA,
        ],
    )
    def sc_kernel(xt_hbm, p_hbm, n_hbm, b_hbm, out_hbm,
                  piv, niv, pv, nv, bv, ov, sem):
        wid = lax.axis_index("s") * _NC + lax.axis_index("c")
        base = wid * bpw

        pltpu.sync_copy(xt_hbm.at[0, pl.ds(base, bpw)], piv)
        pltpu.sync_copy(xt_hbm.at[1, pl.ds(base, bpw)], niv)
        pltpu.sync_copy(b_hbm, bv)

        copies = []
        for c in range(n_ichunks):
            sl = pl.ds(c * _ICHUNK, _ICHUNK)
            copies.append(pltpu.async_copy(p_hbm.at[piv.at[sl]], pv.at[sl], sem))
            copies.append(pltpu.async_copy(n_hbm.at[niv.at[sl]], nv.at[sl], sem))
        for cp in copies:
            cp.wait()

        b_vec = bv[...]

        def add_body(g, carry):
            sl = pl.ds(g * _L, _L)
            ov[sl] = pv[sl] + nv[sl] + b_vec
            return carry
        lax.fori_loop(0, bpw // _L, add_body, 0)

        pltpu.sync_copy(ov, out_hbm.at[pl.ds(base, bpw)])

    return sc_kernel


def kernel(x, emb_proton, emb_neutron, W, b):
    B = x.shape[0]
    R, H = emb_proton.shape
    p, n = _build_tc(R, H, 8192)(emb_proton.T, emb_neutron.T,
                                 W.reshape(1, 2 * H))
    b_vec = jnp.broadcast_to(b.reshape(()), (_L,)).astype(jnp.float32)
    out = _build_sc(B)(x.T, p, n, b_vec)
    return out.reshape(B, 1)


# speedup vs baseline: 4.7688x; 1.0221x over previous
import functools

import jax
import jax.numpy as jnp
from jax import lax
from jax.experimental import pallas as pl
from jax.experimental.pallas import tpu as pltpu
from jax.experimental.pallas import tpu_sc as plsc

_L = 16
_NC = 2
_NS = 16
_NW = _NC * _NS
_ICHUNK = 128


def _tc_body(ptt_ref, ntt_ref, w_ref, b_ref, p_ref, n_ref):
    h = ptt_ref.shape[0]
    br = ptt_ref.shape[1]
    w = w_ref[...]
    dn = (((1,), (0,)), ((), ()))
    p = lax.dot_general(w[:, :h], ptt_ref[...], dn,
                        preferred_element_type=jnp.float32)
    n = lax.dot_general(w[:, h:], ntt_ref[...], dn,
                        preferred_element_type=jnp.float32)
    p_ref[...] = (p + b_ref[0, 0]).reshape(br)
    n_ref[...] = n.reshape(br)


@functools.lru_cache(maxsize=None)
def _build_tc(R, H, BR):
    nblk = (R + BR - 1) // BR
    rpad = nblk * BR
    return pl.pallas_call(
        _tc_body,
        grid=(nblk,),
        in_specs=[
            pl.BlockSpec((H, BR), lambda i: (0, i)),
            pl.BlockSpec((H, BR), lambda i: (0, i)),
            pl.BlockSpec((1, 2 * H), lambda i: (0, 0)),
            pl.BlockSpec((1, 1), lambda i: (0, 0)),
        ],
        out_specs=[
            pl.BlockSpec((BR,), lambda i: (i,)),
            pl.BlockSpec((BR,), lambda i: (i,)),
        ],
        out_shape=[
            jax.ShapeDtypeStruct((rpad,), jnp.float32),
            jax.ShapeDtypeStruct((rpad,), jnp.float32),
        ],
    )


@functools.lru_cache(maxsize=None)
def _build_sc(B):
    bpw = B // _NW
    n_ichunks = bpw // _ICHUNK
    mesh = plsc.VectorSubcoreMesh(core_axis_name="c", subcore_axis_name="s")

    @functools.partial(
        pl.kernel,
        mesh=mesh,
        out_type=jax.ShapeDtypeStruct((B,), jnp.float32),
        compiler_params=pltpu.CompilerParams(
            needs_layout_passes=False, use_tc_tiling_on_sc=False
        ),
        scratch_types=[
            pltpu.VMEM((bpw,), jnp.int32),
            pltpu.VMEM((bpw,), jnp.int32),
            pltpu.VMEM((bpw,), jnp.float32),
            pltpu.VMEM((bpw,), jnp.float32),
            pltpu.VMEM((bpw,), jnp.float32),
            pltpu.SemaphoreType.DMA,
        ],
    )
    def sc_kernel(xt_hbm, p_hbm, n_hbm, out_hbm,
                  piv, niv, pv, nv, ov, sem):
        wid = lax.axis_index("s") * _NC + lax.axis_index("c")
        base = wid * bpw

        pltpu.sync_copy(xt_hbm.at[0, pl.ds(base, bpw)], piv)
        pltpu.sync_copy(xt_hbm.at[1, pl.ds(base, bpw)], niv)

        copies = []
        for c in range(n_ichunks):
            sl = pl.ds(c * _ICHUNK, _ICHUNK)
            copies.append(pltpu.async_copy(p_hbm.at[piv.at[sl]], pv.at[sl], sem))
            copies.append(pltpu.async_copy(n_hbm.at[niv.at[sl]], nv.at[sl], sem))
        for cp in copies:
            cp.wait()

        def add_body(g, carry):
            sl = pl.ds(g * _L, _L)
            ov[sl] = pv[sl] + nv[sl]
            return carry
        lax.fori_loop(0, bpw // _L, add_body, 0)

        pltpu.sync_copy(ov, out_hbm.at[pl.ds(base, bpw)])

    return sc_kernel


def kernel(x, emb_proton, emb_neutron, W, b):
    B = x.shape[0]
    R, H = emb_proton.shape
    p, n = _build_tc(R, H, 8192)(emb_proton.T, emb_neutron.T,
                                 W.reshape(1, 2 * H), b.reshape(1, 1))
    out = _build_sc(B)(x.T, p, n)
    return out.reshape(B, 1)


# speedup vs baseline: 4.9967x; 1.0478x over previous
import functools

import jax
import jax.numpy as jnp
from jax import lax
from jax.experimental import pallas as pl
from jax.experimental.pallas import tpu as pltpu
from jax.experimental.pallas import tpu_sc as plsc

_L = 16
_NC = 2
_NS = 16
_NW = _NC * _NS
_ICHUNK = 128


def _tc_body(ptt_ref, ntt_ref, w_ref, b_ref, p_ref, n_ref):
    h = ptt_ref.shape[0]
    br = ptt_ref.shape[1]
    w = w_ref[...]
    dn = (((1,), (0,)), ((), ()))
    p = lax.dot_general(w[:, :h], ptt_ref[...], dn,
                        preferred_element_type=jnp.float32)
    n = lax.dot_general(w[:, h:], ntt_ref[...], dn,
                        preferred_element_type=jnp.float32)
    p_ref[...] = (p + b_ref[0, 0]).reshape(br)
    n_ref[...] = n.reshape(br)


@functools.lru_cache(maxsize=None)
def _build_tc(R, H, BR):
    nblk = (R + BR - 1) // BR
    rpad = nblk * BR
    return pl.pallas_call(
        _tc_body,
        grid=(nblk,),
        in_specs=[
            pl.BlockSpec((H, BR), lambda i: (0, i)),
            pl.BlockSpec((H, BR), lambda i: (0, i)),
            pl.BlockSpec((1, 2 * H), lambda i: (0, 0)),
            pl.BlockSpec((1, 1), lambda i: (0, 0)),
        ],
        out_specs=[
            pl.BlockSpec((BR,), lambda i: (i,)),
            pl.BlockSpec((BR,), lambda i: (i,)),
        ],
        out_shape=[
            jax.ShapeDtypeStruct((rpad,), jnp.float32),
            jax.ShapeDtypeStruct((rpad,), jnp.float32),
        ],
    )


@functools.lru_cache(maxsize=None)
def _build_sc(B):
    bpw = B // _NW
    n_ichunks = bpw // _ICHUNK
    mesh = plsc.VectorSubcoreMesh(core_axis_name="c", subcore_axis_name="s")

    @functools.partial(
        pl.kernel,
        mesh=mesh,
        out_type=jax.ShapeDtypeStruct((B,), jnp.float32),
        compiler_params=pltpu.CompilerParams(
            needs_layout_passes=False, use_tc_tiling_on_sc=False
        ),
        scratch_types=[
            pltpu.VMEM((bpw,), jnp.int32),
            pltpu.VMEM((bpw,), jnp.int32),
            pltpu.VMEM((bpw,), jnp.float32),
            pltpu.VMEM((bpw,), jnp.float32),
            pltpu.VMEM((bpw,), jnp.float32),
            pltpu.SemaphoreType.DMA,
        ],
    )
    def sc_kernel(xt_hbm, p_hbm, n_hbm, out_hbm,
                  piv, niv, pv, nv, ov, sem):
        wid = lax.axis_index("s") * _NC + lax.axis_index("c")
        base = wid * bpw

        pltpu.sync_copy(xt_hbm.at[0, pl.ds(base, bpw)], piv)
        pltpu.sync_copy(xt_hbm.at[1, pl.ds(base, bpw)], niv)

        copies = []
        for c in range(n_ichunks):
            sl = pl.ds(c * _ICHUNK, _ICHUNK)
            copies.append(pltpu.async_copy(p_hbm.at[piv.at[sl]], pv.at[sl], sem))
            copies.append(pltpu.async_copy(n_hbm.at[niv.at[sl]], nv.at[sl], sem))
        for cp in copies:
            cp.wait()

        def add_body(g, carry):
            sl = pl.ds(g * _L, _L)
            ov[sl] = pv[sl] + nv[sl]
            return carry
        lax.fori_loop(0, bpw // _L, add_body, 0)

        pltpu.sync_copy(ov, out_hbm.at[pl.ds(base, bpw)])

    return sc_kernel


def kernel(x, emb_proton, emb_neutron, W, b):
    B = x.shape[0]
    R, H = emb_proton.shape
    p, n = _build_tc(R, H, 16384)(emb_proton.T, emb_neutron.T,
                                 W.reshape(1, 2 * H), b.reshape(1, 1))
    out = _build_sc(B)(x.T, p, n)
    return out.reshape(B, 1)
